# baseline (speedup 1.0000x reference)
import functools

import jax
import jax.numpy as jnp
from jax import lax
from jax.experimental import pallas as pl
from jax.experimental.pallas import tpu as pltpu
from jax.experimental.pallas import tpu_sc as plsc

NUM_CORES = 2
NUM_SUBCORES = 16
NUM_TILES = NUM_CORES * NUM_SUBCORES
BLK = 128
LANES = 16


GRP = 3
SC_NBLK = 42
XB = 8192
TCB = 8192
def _sc_edge_agg_body(blk0, nblk_full, rem_blocks, num_graphs,
                      edge_attr, dst, batch, zacc,
                      pe, pc,
                      batch_v, seg_flat, seg_slots, rows_all, cnt_v, acc_sh,
                      fsem0, fsem1, ssem0, ssem1):
    c = lax.axis_index("c")
    s = lax.axis_index("s")
    wid = c * NUM_SUBCORES + s
    ngroups = nblk_full // GRP
    fsem = (fsem0, fsem1)
    ssem = (ssem0, ssem1)
    blk_base = blk0 + wid * nblk_full

    def fetch(g, par):
        for j in range(GRP):
            b = g * GRP + j
            slot = par * GRP + j
            pltpu.async_copy(edge_attr.at[pl.ds((blk_base + b) * BLK, BLK)],
                             rows_all.at[pl.ds(slot * BLK, BLK)], fsem[par])

    def wait_fetch(g, par):
        for j in range(GRP):
            b = g * GRP + j
            slot = par * GRP + j
            pltpu.make_async_copy(
                edge_attr.at[pl.ds((blk_base + b) * BLK, BLK)],
                rows_all.at[pl.ds(slot * BLK, BLK)], fsem[par]).wait()

    @pl.when(s == 0)
    def _init():
        pltpu.sync_copy(zacc, acc_sh)

    fetch(0, 0)
    fetch(1, 1)

    pltpu.sync_copy(batch, batch_v)
    for j in range(num_graphs // LANES):
        cnt_v[pl.ds(j * LANES, LANES)] = jnp.zeros((LANES,), jnp.float32)
    pltpu.sync_copy(dst.at[pl.ds(blk_base * BLK, nblk_full * BLK)],
                    seg_flat.at[pl.ds(0, nblk_full * BLK)])
    if rem_blocks:
        @pl.when(wid < rem_blocks)
        def _extra_ids():
            pltpu.sync_copy(
                dst.at[pl.ds((blk0 + NUM_TILES * nblk_full + wid) * BLK, BLK)],
                seg_flat.at[pl.ds(nblk_full * BLK, BLK)])
    plsc.subcore_barrier()

    ones16 = jnp.ones((LANES,), jnp.float32)

    def gather_row(b):
        for k in range(BLK // LANES):
            idx16 = seg_flat[pl.ds(b * BLK + k * LANES, LANES)]
            seg16 = plsc.load_gather(batch_v, [idx16])
            seg_flat[pl.ds(b * BLK + k * LANES, LANES)] = seg16
            plsc.addupdate_scatter(cnt_v, [seg16], ones16)

    def gather_body(b, carry):
        gather_row(b)
        return carry

    lax.fori_loop(0, nblk_full, gather_body, 0)
    if rem_blocks:
        @pl.when(wid < rem_blocks)
        def _extra_gather():
            gather_row(nblk_full)

    def do_group(g, par, do_fetch):
        wait_fetch(g, par)
        for j in range(GRP):
            b = g * GRP + j
            slot = par * GRP + j
            for k in range(BLK // LANES):
                seg_slots[slot, pl.ds(k * LANES, LANES)] = (
                    seg_flat[pl.ds(b * BLK + k * LANES, LANES)])
        descs = []
        for j in range(GRP):
            slot = par * GRP + j
            descs.append(pltpu.async_copy(
                rows_all.at[pl.ds(slot * BLK, BLK)],
                acc_sh.at[seg_slots.at[slot]], ssem[par], add=True))
        for d in descs:
            d.wait()
        if do_fetch:
            fetch(g + 2, par)

    def pipe_body(i, carry):
        do_group(2 * i, 0, True)
        do_group(2 * i + 1, 1, True)
        return carry

    lax.fori_loop(0, ngroups // 2 - 1, pipe_body, 0)
    do_group(ngroups - 2, 0, False)
    do_group(ngroups - 1, 1, False)

    if rem_blocks:
        @pl.when(wid < rem_blocks)
        def _extra():
            off = (blk0 + NUM_TILES * nblk_full + wid) * BLK
            for k in range(BLK // LANES):
                seg_slots[0, pl.ds(k * LANES, LANES)] = (
                    seg_flat[pl.ds(nblk_full * BLK + k * LANES, LANES)])
            pltpu.sync_copy(edge_attr.at[pl.ds(off, BLK)],
                            rows_all.at[pl.ds(0, BLK)])
            pltpu.sync_copy(rows_all.at[pl.ds(0, BLK)],
                            acc_sh.at[seg_slots.at[0]], add=True)

    pltpu.sync_copy(cnt_v, pc.at[pl.ds(wid * num_graphs, num_graphs)])

    plsc.subcore_barrier()

    @pl.when(s == 0)
    def _writeout():
        pltpu.sync_copy(acc_sh, pe.at[c])


def _make_sc_edge_agg(blk0, n_sc_edges, n_nodes, num_graphs, hidden):
    assert n_sc_edges % BLK == 0
    nblk = n_sc_edges // BLK
    nblk_full = nblk // NUM_TILES
    nblk_full -= nblk_full % (2 * GRP)
    rem_blocks = nblk - nblk_full * NUM_TILES
    assert rem_blocks <= NUM_TILES
    mesh = plsc.VectorSubcoreMesh(core_axis_name="c", subcore_axis_name="s")
    return pl.kernel(
        functools.partial(_sc_edge_agg_body, blk0, nblk_full, rem_blocks,
                          num_graphs),
        out_type=(
            jax.ShapeDtypeStruct((NUM_CORES, num_graphs, hidden), jnp.float32),
            jax.ShapeDtypeStruct((NUM_TILES * num_graphs,), jnp.float32),
        ),
        mesh=mesh,
        compiler_params=pltpu.CompilerParams(needs_layout_passes=False),
        scratch_types=[
            pltpu.VMEM((n_nodes,), jnp.int32),
            pltpu.VMEM(((nblk_full + 1) * BLK,), jnp.int32),
            pltpu.VMEM((2 * GRP, BLK), jnp.int32),
            pltpu.VMEM((2 * GRP * BLK, hidden), jnp.float32),
            pltpu.VMEM((num_graphs,), jnp.float32),
            pltpu.VMEM_SHARED((num_graphs, hidden), jnp.float32),
            pltpu.SemaphoreType.DMA,
            pltpu.SemaphoreType.DMA,
            pltpu.SemaphoreType.DMA,
            pltpu.SemaphoreType.DMA,
        ],
    )


def _tc_extract_body(ei_ref, o_ref):
    o_ref[...] = ei_ref[1, :]


def _tc_node_body(num_graphs, x_ref, b_ref, nm_ref, st_ref):
    n = x_ref.shape[0]
    seg = jnp.broadcast_to(b_ref[...], (num_graphs, n))
    gid = lax.broadcasted_iota(jnp.int32, (num_graphs, n), 0)
    oh = (seg == gid).astype(jnp.float32)
    nsum = jax.lax.dot(oh, x_ref[...], precision=jax.lax.Precision.HIGHEST,
                       preferred_element_type=jnp.float32)
    ncnt = jnp.sum(oh, axis=1, keepdims=True)
    nm_ref[...] = nsum / jnp.maximum(ncnt, 1.0)
    gi = lax.broadcasted_iota(jnp.int32, (num_graphs, num_graphs), 0)
    gj = lax.broadcasted_iota(jnp.int32, (num_graphs, num_graphs), 1)
    ltri = (gj < gi).astype(jnp.float32)
    st_ref[...] = jax.lax.dot(ltri, ncnt,
                              precision=jax.lax.Precision.HIGHEST,
                              preferred_element_type=jnp.float32)


def _tc_edge_body(num_graphs, e_sc, n_edges, ea_ref, ei_ref, st_ref,
                  es_ref, ec_ref):
    i = pl.program_id(0)

    @pl.when(i == 0)
    def _zero():
        es_ref[...] = jnp.zeros_like(es_ref)
        ec_ref[...] = jnp.zeros_like(ec_ref)

    n = ea_ref.shape[0]
    gbase = e_sc + i * n
    valid_c = (lax.broadcasted_iota(jnp.int32, (n, 1), 0) + gbase) < n_edges
    valid_r = ((lax.broadcasted_iota(jnp.int32, (1, n), 1) + gbase)
               < n_edges).astype(jnp.float32)
    ea = jnp.where(valid_c, ea_ref[...], 0.0)
    dstf = ei_ref[1:2, :].astype(jnp.float32)
    cmp = (st_ref[...] <= dstf).astype(jnp.float32)
    eseg = (jnp.sum(cmp, axis=0, keepdims=True) - 1.0).astype(jnp.int32)
    oh = (lax.broadcasted_iota(jnp.int32, (num_graphs, n), 0)
          == eseg).astype(jnp.float32) * valid_r
    es_ref[...] += jax.lax.dot(oh, ea,
                               precision=jax.lax.Precision.HIGHEST,
                               preferred_element_type=jnp.float32)
    ec_ref[...] += jnp.sum(oh, axis=1, keepdims=True)


def _tc_final_body(num_graphs, nm_ref, u_ref, pe_ref, pc_ref, es_ref, ec_ref,
                   w1u_ref, w1n_ref, w1e_ref, b1_ref, w2_ref, b2_ref, o_ref):
    esum = pe_ref[0] + pe_ref[1] + es_ref[...]
    ecnt = lax.dot_general(pc_ref[...], jnp.ones((NUM_TILES, 1), jnp.float32),
                           dimension_numbers=(((0,), (0,)), ((), ())),
                           precision=jax.lax.Precision.HIGHEST,
                           preferred_element_type=jnp.float32) + ec_ref[...]
    emean = esum / jnp.maximum(ecnt, 1.0)
    dot = functools.partial(jax.lax.dot, precision=jax.lax.Precision.HIGHEST,
                            preferred_element_type=jnp.float32)
    h = jnp.maximum(
        dot(u_ref[...], w1u_ref[...]) + dot(nm_ref[...], w1n_ref[...])
        + dot(emean, w1e_ref[...]) + b1_ref[...], 0.0)
    o_ref[...] = dot(h, w2_ref[...]) + b2_ref[...]


def kernel(x, edge_index, edge_attr, u, batch, W1, b1, W2, b2):
    n_nodes, hidden = x.shape
    n_edges = edge_attr.shape[0]
    num_graphs, u_in = u.shape
    e_sc = NUM_TILES * SC_NBLK * BLK
    e_tc = n_edges - e_sc
    assert e_sc % XB == 0 and e_sc % TCB == 0

    batch32 = batch.astype(jnp.int32)
    ei32 = edge_index.astype(jnp.int32)
    zacc = jnp.zeros((num_graphs, hidden), jnp.float32)

    extract = pl.pallas_call(
        _tc_extract_body,
        grid=(e_sc // XB,),
        in_specs=[pl.BlockSpec((2, XB), lambda i: (0, i))],
        out_specs=pl.BlockSpec((XB,), lambda i: (i,)),
        out_shape=jax.ShapeDtypeStruct((e_sc,), jnp.int32),
    )
    dst = extract(ei32)

    sc_agg = _make_sc_edge_agg(0, e_sc, n_nodes, num_graphs, hidden)
    pe, pc = sc_agg(edge_attr, dst, batch32, zacc)
    pc = pc.reshape(NUM_TILES, num_graphs)

    tc_node = pl.pallas_call(
        functools.partial(_tc_node_body, num_graphs),
        out_shape=(jax.ShapeDtypeStruct((num_graphs, hidden), jnp.float32),
                   jax.ShapeDtypeStruct((num_graphs, 1), jnp.float32)),
    )
    nmean, starts = tc_node(x, batch32.reshape(1, n_nodes))

    sc_blk = e_sc // TCB
    tc_edge = pl.pallas_call(
        functools.partial(_tc_edge_body, num_graphs, e_sc, n_edges),
        grid=((e_tc + TCB - 1) // TCB,),
        in_specs=[pl.BlockSpec((TCB, hidden), lambda i: (sc_blk + i, 0)),
                  pl.BlockSpec((2, TCB), lambda i: (0, sc_blk + i)),
                  pl.BlockSpec((num_graphs, 1), lambda i: (0, 0))],
        out_specs=(pl.BlockSpec((num_graphs, hidden), lambda i: (0, 0)),
                   pl.BlockSpec((num_graphs, 1), lambda i: (0, 0))),
        out_shape=(jax.ShapeDtypeStruct((num_graphs, hidden), jnp.float32),
                   jax.ShapeDtypeStruct((num_graphs, 1), jnp.float32)),
    )
    es_tc, ec_tc = tc_edge(edge_attr, ei32, starts)

    w1u_t = W1[:, :u_in].T
    w1n_t = W1[:, u_in:u_in + hidden].T
    w1e_t = W1[:, u_in + hidden:].T
    tc_final = pl.pallas_call(
        functools.partial(_tc_final_body, num_graphs),
        out_shape=jax.ShapeDtypeStruct((num_graphs, hidden), jnp.float32),
    )
    return tc_final(nmean, u, pe, pc, es_tc, ec_tc,
                    w1u_t, w1n_t, w1e_t, b1.reshape(1, hidden),
                    W2.T, b2.reshape(1, hidden))


# speedup vs baseline: 1.2535x; 1.2535x over previous
import functools

import jax
import jax.numpy as jnp
from jax import lax
from jax.experimental import pallas as pl
from jax.experimental.pallas import tpu as pltpu
from jax.experimental.pallas import tpu_sc as plsc

NUM_CORES = 2
NUM_SUBCORES = 16
NUM_TILES = NUM_CORES * NUM_SUBCORES
BLK = 128
LANES = 16


GRP = 3
SC_NBLK = 42
XB = 131072
TCB = 8192
NODE_B = 2000
def _sc_edge_agg_body(blk0, extra_blk0, nblk_full, rem_blocks, num_graphs,
                      edge_attr, dst, batch, zacc,
                      pe, pc,
                      batch_v, seg_flat, seg_slots, rows_all, cnt_v, acc_sh,
                      fsem0, fsem1, ssem0, ssem1):
    c = lax.axis_index("c")
    s = lax.axis_index("s")
    wid = c * NUM_SUBCORES + s
    ngroups = nblk_full // GRP
    fsem = (fsem0, fsem1)
    ssem = (ssem0, ssem1)
    blk_base = blk0 + wid * nblk_full

    def fetch(g, par):
        for j in range(GRP):
            b = g * GRP + j
            slot = par * GRP + j
            pltpu.async_copy(edge_attr.at[pl.ds((blk_base + b) * BLK, BLK)],
                             rows_all.at[pl.ds(slot * BLK, BLK)], fsem[par])

    def wait_fetch(g, par):
        for j in range(GRP):
            b = g * GRP + j
            slot = par * GRP + j
            pltpu.make_async_copy(
                edge_attr.at[pl.ds((blk_base + b) * BLK, BLK)],
                rows_all.at[pl.ds(slot * BLK, BLK)], fsem[par]).wait()

    @pl.when(s == 0)
    def _init():
        pltpu.sync_copy(zacc, acc_sh)

    fetch(0, 0)
    fetch(1, 1)

    pltpu.sync_copy(batch, batch_v)
    for j in range(num_graphs // LANES):
        cnt_v[pl.ds(j * LANES, LANES)] = jnp.zeros((LANES,), jnp.float32)
    pltpu.sync_copy(dst.at[pl.ds(blk_base * BLK, nblk_full * BLK)],
                    seg_flat.at[pl.ds(0, nblk_full * BLK)])
    if rem_blocks:
        @pl.when(wid < rem_blocks)
        def _extra_ids():
            pltpu.sync_copy(
                dst.at[pl.ds((extra_blk0 + wid) * BLK, BLK)],
                seg_flat.at[pl.ds(nblk_full * BLK, BLK)])
    plsc.subcore_barrier()

    ones16 = jnp.ones((LANES,), jnp.float32)

    def gather_row(b):
        for k in range(BLK // LANES):
            idx16 = seg_flat[pl.ds(b * BLK + k * LANES, LANES)]
            seg16 = plsc.load_gather(batch_v, [idx16])
            seg_flat[pl.ds(b * BLK + k * LANES, LANES)] = seg16
            plsc.addupdate_scatter(cnt_v, [seg16], ones16)

    def gather_body(b, carry):
        gather_row(b)
        return carry

    lax.fori_loop(0, nblk_full, gather_body, 0)
    if rem_blocks:
        @pl.when(wid < rem_blocks)
        def _extra_gather():
            gather_row(nblk_full)

    def do_group(g, par, do_fetch):
        wait_fetch(g, par)
        for j in range(GRP):
            b = g * GRP + j
            slot = par * GRP + j
            for k in range(BLK // LANES):
                seg_slots[slot, pl.ds(k * LANES, LANES)] = (
                    seg_flat[pl.ds(b * BLK + k * LANES, LANES)])
        descs = []
        for j in range(GRP):
            slot = par * GRP + j
            descs.append(pltpu.async_copy(
                rows_all.at[pl.ds(slot * BLK, BLK)],
                acc_sh.at[seg_slots.at[slot]], ssem[par], add=True))
        for d in descs:
            d.wait()
        if do_fetch:
            fetch(g + 2, par)

    def pipe_body(i, carry):
        do_group(2 * i, 0, True)
        do_group(2 * i + 1, 1, True)
        return carry

    lax.fori_loop(0, ngroups // 2 - 1, pipe_body, 0)
    do_group(ngroups - 2, 0, False)
    do_group(ngroups - 1, 1, False)

    if rem_blocks:
        @pl.when(wid < rem_blocks)
        def _extra():
            off = (extra_blk0 + wid) * BLK
            for k in range(BLK // LANES):
                seg_slots[0, pl.ds(k * LANES, LANES)] = (
                    seg_flat[pl.ds(nblk_full * BLK + k * LANES, LANES)])
            pltpu.sync_copy(edge_attr.at[pl.ds(off, BLK)],
                            rows_all.at[pl.ds(0, BLK)])
            pltpu.sync_copy(rows_all.at[pl.ds(0, BLK)],
                            acc_sh.at[seg_slots.at[0]], add=True)

    pltpu.sync_copy(cnt_v, pc.at[pl.ds(wid * num_graphs, num_graphs)])

    plsc.subcore_barrier()

    @pl.when(s == 0)
    def _writeout():
        pltpu.sync_copy(acc_sh, pe.at[c])


def _make_sc_edge_agg(blk0, extra_blk0, n_sc_edges, n_nodes, num_graphs,
                      hidden):
    assert n_sc_edges % BLK == 0
    nblk = n_sc_edges // BLK
    nblk_full = nblk // NUM_TILES
    nblk_full -= nblk_full % (2 * GRP)
    rem_blocks = nblk - nblk_full * NUM_TILES
    assert rem_blocks <= NUM_TILES
    mesh = plsc.VectorSubcoreMesh(core_axis_name="c", subcore_axis_name="s")
    return pl.kernel(
        functools.partial(_sc_edge_agg_body, blk0, extra_blk0, nblk_full,
                          rem_blocks, num_graphs),
        out_type=(
            jax.ShapeDtypeStruct((NUM_CORES, num_graphs, hidden), jnp.float32),
            jax.ShapeDtypeStruct((NUM_TILES * num_graphs,), jnp.float32),
        ),
        mesh=mesh,
        compiler_params=pltpu.CompilerParams(needs_layout_passes=False),
        scratch_types=[
            pltpu.VMEM((n_nodes,), jnp.int32),
            pltpu.VMEM(((nblk_full + 1) * BLK,), jnp.int32),
            pltpu.VMEM((2 * GRP, BLK), jnp.int32),
            pltpu.VMEM((2 * GRP * BLK, hidden), jnp.float32),
            pltpu.VMEM((num_graphs,), jnp.float32),
            pltpu.VMEM_SHARED((num_graphs, hidden), jnp.float32),
            pltpu.SemaphoreType.DMA,
            pltpu.SemaphoreType.DMA,
            pltpu.SemaphoreType.DMA,
            pltpu.SemaphoreType.DMA,
        ],
    )


def _tc_extract_body(ei_ref, o_ref):
    o_ref[...] = ei_ref[1, :]


def _tc_node_body(num_graphs, nsteps, x_ref, b_ref, nm_ref, st_ref, cnt_s):
    i = pl.program_id(0)
    n = x_ref.shape[0]

    @pl.when(i == 0)
    def _zero():
        nm_ref[...] = jnp.zeros_like(nm_ref)
        cnt_s[...] = jnp.zeros_like(cnt_s)

    seg = jnp.broadcast_to(b_ref[0], (num_graphs, n))
    gid = lax.broadcasted_iota(jnp.int32, (num_graphs, n), 0)
    oh = (seg == gid).astype(jnp.float32)
    nm_ref[...] += jax.lax.dot(oh, x_ref[...],
                               precision=jax.lax.Precision.HIGHEST,
                               preferred_element_type=jnp.float32)
    cnt_s[...] += jnp.sum(oh, axis=1, keepdims=True)

    @pl.when(i == nsteps - 1)
    def _finish():
        ncnt = cnt_s[...]
        nm_ref[...] = nm_ref[...] / jnp.maximum(ncnt, 1.0)
        gi = lax.broadcasted_iota(jnp.int32, (num_graphs, num_graphs), 0)
        gj = lax.broadcasted_iota(jnp.int32, (num_graphs, num_graphs), 1)
        ltri = (gj < gi).astype(jnp.float32)
        st_ref[...] = jax.lax.dot(ltri, ncnt,
                                  precision=jax.lax.Precision.HIGHEST,
                                  preferred_element_type=jnp.float32)


def _tc_edge_body(num_graphs, ea_ref, ei_ref, st_ref, es_ref, ec_ref):
    i = pl.program_id(0)

    @pl.when(i == 0)
    def _zero():
        es_ref[...] = jnp.zeros_like(es_ref)
        ec_ref[...] = jnp.zeros_like(ec_ref)

    n = ea_ref.shape[0]
    dstf = ei_ref[1:2, :].astype(jnp.float32)
    cmp = (st_ref[...] <= dstf).astype(jnp.float32)
    eseg = (jnp.sum(cmp, axis=0, keepdims=True) - 1.0).astype(jnp.int32)
    oh = (lax.broadcasted_iota(jnp.int32, (num_graphs, n), 0)
          == eseg).astype(jnp.float32)
    es_ref[...] += jax.lax.dot(oh, ea_ref[...],
                               precision=jax.lax.Precision.HIGHEST,
                               preferred_element_type=jnp.float32)
    ec_ref[...] += jnp.sum(oh, axis=1, keepdims=True)


def _tc_final_body(num_graphs, nm_ref, u_ref, pe_ref, pc_ref, es_ref, ec_ref,
                   w1u_ref, w1n_ref, w1e_ref, b1_ref, w2_ref, b2_ref, o_ref):
    esum = pe_ref[0] + pe_ref[1] + es_ref[...]
    ecnt = lax.dot_general(pc_ref[...], jnp.ones((NUM_TILES, 1), jnp.float32),
                           dimension_numbers=(((0,), (0,)), ((), ())),
                           precision=jax.lax.Precision.HIGHEST,
                           preferred_element_type=jnp.float32) + ec_ref[...]
    emean = esum / jnp.maximum(ecnt, 1.0)
    dot = functools.partial(jax.lax.dot, precision=jax.lax.Precision.HIGHEST,
                            preferred_element_type=jnp.float32)
    h = jnp.maximum(
        dot(u_ref[...], w1u_ref[...]) + dot(nm_ref[...], w1n_ref[...])
        + dot(emean, w1e_ref[...]) + b1_ref[...], 0.0)
    o_ref[...] = dot(h, w2_ref[...]) + b2_ref[...]


def kernel(x, edge_index, edge_attr, u, batch, W1, b1, W2, b2):
    n_nodes, hidden = x.shape
    n_edges = edge_attr.shape[0]
    num_graphs, u_in = u.shape
    e_sc = NUM_TILES * SC_NBLK * BLK
    e_tc = (n_edges - e_sc) // TCB * TCB
    e_rem = n_edges - e_sc - e_tc
    assert e_sc % TCB == 0 and e_rem % BLK == 0 and e_rem // BLK <= NUM_TILES

    batch32 = batch.astype(jnp.int32)
    ei32 = edge_index.astype(jnp.int32)
    zacc = jnp.zeros((num_graphs, hidden), jnp.float32)

    extract = pl.pallas_call(
        _tc_extract_body,
        grid=((n_edges + XB - 1) // XB,),
        in_specs=[pl.BlockSpec((2, XB), lambda i: (0, i))],
        out_specs=pl.BlockSpec((XB,), lambda i: (i,)),
        out_shape=jax.ShapeDtypeStruct((n_edges,), jnp.int32),
    )
    dst = extract(ei32)

    sc_agg = _make_sc_edge_agg(0, (e_sc + e_tc) // BLK, e_sc + e_rem,
                               n_nodes, num_graphs, hidden)
    pe, pc = sc_agg(edge_attr, dst, batch32, zacc)
    pc = pc.reshape(NUM_TILES, num_graphs)

    assert n_nodes % NODE_B == 0
    tc_node = pl.pallas_call(
        functools.partial(_tc_node_body, num_graphs, n_nodes // NODE_B),
        grid=(n_nodes // NODE_B,),
        in_specs=[pl.BlockSpec((NODE_B, hidden), lambda i: (i, 0)),
                  pl.BlockSpec((1, 1, NODE_B), lambda i: (i, 0, 0))],
        out_specs=(pl.BlockSpec((num_graphs, hidden), lambda i: (0, 0)),
                   pl.BlockSpec((num_graphs, 1), lambda i: (0, 0))),
        out_shape=(jax.ShapeDtypeStruct((num_graphs, hidden), jnp.float32),
                   jax.ShapeDtypeStruct((num_graphs, 1), jnp.float32)),
        scratch_shapes=[pltpu.VMEM((num_graphs, 1), jnp.float32)],
    )
    nmean, starts = tc_node(x, batch32.reshape(n_nodes // NODE_B, 1, NODE_B))

    sc_blk = e_sc // TCB
    tc_edge = pl.pallas_call(
        functools.partial(_tc_edge_body, num_graphs),
        grid=(e_tc // TCB,),
        in_specs=[pl.BlockSpec((TCB, hidden), lambda i: (sc_blk + i, 0)),
                  pl.BlockSpec((2, TCB), lambda i: (0, sc_blk + i)),
                  pl.BlockSpec((num_graphs, 1), lambda i: (0, 0))],
        out_specs=(pl.BlockSpec((num_graphs, hidden), lambda i: (0, 0)),
                   pl.BlockSpec((num_graphs, 1), lambda i: (0, 0))),
        out_shape=(jax.ShapeDtypeStruct((num_graphs, hidden), jnp.float32),
                   jax.ShapeDtypeStruct((num_graphs, 1), jnp.float32)),
    )
    es_tc, ec_tc = tc_edge(edge_attr, ei32, starts)

    w1u_t = W1[:, :u_in].T
    w1n_t = W1[:, u_in:u_in + hidden].T
    w1e_t = W1[:, u_in + hidden:].T
    tc_final = pl.pallas_call(
        functools.partial(_tc_final_body, num_graphs),
        out_shape=jax.ShapeDtypeStruct((num_graphs, hidden), jnp.float32),
    )
    return tc_final(nmean, u, pe, pc, es_tc, ec_tc,
                    w1u_t, w1n_t, w1e_t, b1.reshape(1, hidden),
                    W2.T, b2.reshape(1, hidden))
